# baseline (speedup 1.0000x reference)
import functools

import jax
import jax.numpy as jnp
from jax.experimental import pallas as pl
from jax.experimental.pallas import tpu as pltpu


_BB = 32


def _wnorm(v, g):
    norm = jnp.sqrt(jnp.sum(v * v, axis=(1, 2, 3), keepdims=True))
    return v * (g.reshape(-1, 1, 1, 1) / norm)


def _band_rhs(v, g, w_sp, mxu_dtype):
    wt = _wnorm(v, g).transpose(2, 3, 1, 0)
    cin, cout = wt.shape[2], wt.shape[3]
    wp = jnp.arange(w_sp)
    s = (wp[None, :, None] - wp[None, None, :] + 1
         == jnp.arange(3)[:, None, None]).astype(jnp.float32)
    r = jnp.einsum('xuw,hxio->huiwo', s, wt)
    perm = jnp.concatenate([jnp.arange(0, w_sp, 2), jnp.arange(1, w_sp, 2)])
    r = r[:, :, :, perm, :]
    return r.reshape(3 * w_sp * cin, w_sp * cout).astype(mxu_dtype)


def _fused_body(x_ref,
                r1_ref, b1_ref, r2_ref, b2_ref, r3_ref, b3_ref,
                fw_ref, fb_ref,
                o_ref,
                p1_ref, l1_ref, s1_ref, p2_ref, l2_ref, s2_ref,
                p3_ref, l3_ref, s3_ref):
    B, H, WC = x_ref.shape

    def stage(p_ref, l_ref, s_ref, r_ref, b_ref, h_sp, sink):
        wc = p_ref.shape[-1]
        slabs = [p_ref[:, k:k + h_sp, :].astype(r_ref.dtype)
                 .reshape(B * h_sp, wc) for k in range(3)]
        nch = s_ref.shape[2]
        half = nch * 128
        acc_e = sum(jnp.dot(s, r_ref[k * wc:(k + 1) * wc, :half],
                            preferred_element_type=jnp.float32)
                    for k, s in enumerate(slabs))
        acc_o = sum(jnp.dot(s, r_ref[k * wc:(k + 1) * wc, half:],
                            preferred_element_type=jnp.float32)
                    for k, s in enumerate(slabs))
        y = jnp.maximum(jnp.maximum(acc_e, acc_o) + b_ref[:, :half], 0.0)
        for c in range(nch):
            s_ref[:, :, c, :] = y[:, c * 128:(c + 1) * 128].reshape(
                B, h_sp, 128)
        for c in range(nch):
            sink(c, jnp.maximum(s_ref[:, 0::2, c, :], s_ref[:, 1::2, c, :]))

    def fill_pad(p_ref, h_sp):
        z = jnp.zeros((B, 1, p_ref.shape[-1]), p_ref.dtype)
        p_ref[:, 0:1, :] = z
        p_ref[:, h_sp + 1:h_sp + 2, :] = z

    fill_pad(p1_ref, 32)
    p1_ref[:, 1:33, :] = x_ref[...].astype(p1_ref.dtype)
    fill_pad(p2_ref, 16)
    stage(p1_ref, l1_ref, s1_ref, r1_ref, b1_ref, 32,
          lambda c, v: p2_ref.__setitem__(
              (slice(None), slice(1, 17), slice(c * 128, (c + 1) * 128)),
              v.astype(p2_ref.dtype)))
    fill_pad(p3_ref, 8)
    stage(p2_ref, l2_ref, s2_ref, r2_ref, b2_ref, 16,
          lambda c, v: p3_ref.__setitem__(
              (slice(None), slice(1, 9), slice(c * 128, (c + 1) * 128)),
              v.astype(p3_ref.dtype)))

    feats = [None] * s3_ref.shape[2]

    def head_sink(c, v):
        feats[c] = (v[:, 0] + v[:, 1] + v[:, 2] + v[:, 3]) * (1.0 / 16.0)

    stage(p3_ref, l3_ref, s3_ref, r3_ref, b3_ref, 8, head_sink)

    logits = fb_ref[...]
    for c, feat in enumerate(feats):
        logits = logits + jnp.dot(feat.astype(fw_ref.dtype),
                                  fw_ref[c * 128:(c + 1) * 128, :],
                                  preferred_element_type=jnp.float32)
    o_ref[...] = logits.astype(o_ref.dtype)


@functools.partial(jax.jit, static_argnames=("batch_block", "mxu_dtype"))
def _forward(x_nchw, v0, g0, b0, v1, g1, b1, v2, g2, b2, fc_w, fc_b,
             *, batch_block=_BB, mxu_dtype=jnp.bfloat16):
    N, Cin, H, W = x_nchw.shape
    B = batch_block

    x = jnp.transpose(x_nchw, (0, 2, 3, 1)).reshape(N, H, W * Cin)
    n_pad = (-N) % B
    if n_pad:
        x = jnp.concatenate(
            [x, jnp.zeros((n_pad, H, W * Cin), x.dtype)], axis=0)
    Np = N + n_pad

    c1, c2, c3 = v0.shape[0], v1.shape[0], v2.shape[0]
    w1, w2, w3 = W, W // 2, W // 4
    r1 = _band_rhs(v0, g0, w1, mxu_dtype)
    r2 = _band_rhs(v1, g1, w2, mxu_dtype)
    r3 = _band_rhs(v2, g2, w3, mxu_dtype)
    b1t = jnp.tile(b0.astype(jnp.float32), w1).reshape(1, w1 * c1)
    b2t = jnp.tile(b1.astype(jnp.float32), w2).reshape(1, w2 * c2)
    b3t = jnp.tile(b2.astype(jnp.float32), w3).reshape(1, w3 * c3)

    nc = fc_w.shape[0]
    nc_pad = max(128, ((nc + 127) // 128) * 128)
    w4 = W // 8
    fp = jnp.zeros((c3, nc_pad), mxu_dtype).at[:, :nc].set(
        fc_w.T.astype(mxu_dtype))
    fw = jnp.tile(fp, (w4, 1))
    fb = jnp.zeros((1, nc_pad), jnp.float32).at[:, :nc].set(
        fc_b.astype(jnp.float32))

    h2, h3 = H // 2, H // 4
    out = pl.pallas_call(
        _fused_body,
        out_shape=jax.ShapeDtypeStruct((Np, nc_pad), jnp.float32),
        grid_spec=pltpu.PrefetchScalarGridSpec(
            num_scalar_prefetch=0,
            grid=(Np // B,),
            in_specs=[
                pl.BlockSpec((B, H, W * Cin), lambda n: (n, 0, 0)),
                pl.BlockSpec(r1.shape, lambda n: (0, 0)),
                pl.BlockSpec(b1t.shape, lambda n: (0, 0)),
                pl.BlockSpec(r2.shape, lambda n: (0, 0)),
                pl.BlockSpec(b2t.shape, lambda n: (0, 0)),
                pl.BlockSpec(r3.shape, lambda n: (0, 0)),
                pl.BlockSpec(b3t.shape, lambda n: (0, 0)),
                pl.BlockSpec(fw.shape, lambda n: (0, 0)),
                pl.BlockSpec(fb.shape, lambda n: (0, 0)),
            ],
            out_specs=pl.BlockSpec((B, nc_pad), lambda n: (n, 0)),
            scratch_shapes=[
                pltpu.VMEM((B, H + 2, W * Cin), jnp.float32),
                pltpu.VMEM((B, H, 3 * W * Cin), mxu_dtype),
                pltpu.VMEM((B, H, (w2 * c1) // 128, 128),
                           jnp.float32),
                pltpu.VMEM((B, h2 + 2, w2 * c1), jnp.float32),
                pltpu.VMEM((B, h2, 3 * w2 * c1), mxu_dtype),
                pltpu.VMEM((B, h2, (w3 * c2) // 128, 128),
                           jnp.float32),
                pltpu.VMEM((B, h3 + 2, w3 * c2), jnp.float32),
                pltpu.VMEM((B, h3, 3 * w3 * c2), mxu_dtype),
                pltpu.VMEM((B, h3, ((W // 8) * c3) // 128, 128),
                           jnp.float32),
            ],
        ),
        compiler_params=pltpu.CompilerParams(
            dimension_semantics=("parallel",)),
    )(x, r1, b1t, r2, b2t, r3, b3t, fw, fb)
    return out[:N, :nc]


def kernel(x, v0, g0, b0, v1, g1, b1, v2, g2, b2, fc_w, fc_b):
    return _forward(x, v0, g0, b0, v1, g1, b1, v2, g2, b2, fc_w, fc_b)


# speedup vs baseline: 1.0389x; 1.0389x over previous
import functools

import jax
import jax.numpy as jnp
from jax.experimental import pallas as pl
from jax.experimental.pallas import tpu as pltpu


_BB = 32


def _wnorm(v, g):
    norm = jnp.sqrt(jnp.sum(v * v, axis=(1, 2, 3), keepdims=True))
    return v * (g.reshape(-1, 1, 1, 1) / norm)


def _band_rhs(v, g, w_sp, mxu_dtype):
    wt = _wnorm(v, g).transpose(2, 3, 1, 0)
    cin, cout = wt.shape[2], wt.shape[3]
    wp = jnp.arange(w_sp)
    s = (wp[None, :, None] - wp[None, None, :] + 1
         == jnp.arange(3)[:, None, None]).astype(jnp.float32)
    r = jnp.einsum('xuw,hxio->huiwo', s, wt)
    perm = jnp.concatenate([jnp.arange(0, w_sp, 2), jnp.arange(1, w_sp, 2)])
    r = r[:, :, :, perm, :]
    return r.reshape(3 * w_sp * cin, w_sp * cout).astype(mxu_dtype)


def _fused_body(x_ref,
                r1_ref, b1_ref, r2_ref, b2_ref, r3_ref, b3_ref,
                fw_ref, fb_ref,
                o_ref,
                p1_ref, l1_ref, s1_ref, p2_ref, l2_ref, s2_ref,
                p3_ref, l3_ref, s3_ref):
    B, H, WC = x_ref.shape

    def stage(p_ref, l_ref, s_ref, r_ref, b_ref, h_sp, sink):
        wc = p_ref.shape[-1]
        for k in range(3):
            l_ref[:, :, k * wc:(k + 1) * wc] = (
                p_ref[:, k:k + h_sp, :].astype(l_ref.dtype))
        lhs = l_ref[...].reshape(B * h_sp, 3 * wc)
        nch = s_ref.shape[2]
        half = nch * 128
        acc_e = jnp.dot(lhs, r_ref[:, :half],
                        preferred_element_type=jnp.float32)
        acc_o = jnp.dot(lhs, r_ref[:, half:],
                        preferred_element_type=jnp.float32)
        y = jnp.maximum(jnp.maximum(acc_e, acc_o) + b_ref[:, :half], 0.0)
        for c in range(nch):
            s_ref[:, :, c, :] = y[:, c * 128:(c + 1) * 128].reshape(
                B, h_sp, 128)
        for c in range(nch):
            sink(c, jnp.maximum(s_ref[:, 0::2, c, :], s_ref[:, 1::2, c, :]))

    def fill_pad(p_ref, h_sp):
        z = jnp.zeros((B, 1, p_ref.shape[-1]), p_ref.dtype)
        p_ref[:, 0:1, :] = z
        p_ref[:, h_sp + 1:h_sp + 2, :] = z

    fill_pad(p1_ref, 32)
    p1_ref[:, 1:33, :] = x_ref[...].astype(p1_ref.dtype)
    fill_pad(p2_ref, 16)
    stage(p1_ref, l1_ref, s1_ref, r1_ref, b1_ref, 32,
          lambda c, v: p2_ref.__setitem__(
              (slice(None), slice(1, 17), slice(c * 128, (c + 1) * 128)),
              v.astype(p2_ref.dtype)))
    fill_pad(p3_ref, 8)
    stage(p2_ref, l2_ref, s2_ref, r2_ref, b2_ref, 16,
          lambda c, v: p3_ref.__setitem__(
              (slice(None), slice(1, 9), slice(c * 128, (c + 1) * 128)),
              v.astype(p3_ref.dtype)))

    feats = [None] * s3_ref.shape[2]

    def head_sink(c, v):
        feats[c] = (v[:, 0] + v[:, 1] + v[:, 2] + v[:, 3]) * (1.0 / 16.0)

    stage(p3_ref, l3_ref, s3_ref, r3_ref, b3_ref, 8, head_sink)

    logits = fb_ref[...]
    for c, feat in enumerate(feats):
        logits = logits + jnp.dot(feat.astype(fw_ref.dtype),
                                  fw_ref[c * 128:(c + 1) * 128, :],
                                  preferred_element_type=jnp.float32)
    o_ref[...] = logits.astype(o_ref.dtype)


@functools.partial(jax.jit, static_argnames=("batch_block", "mxu_dtype"))
def _forward(x_nchw, v0, g0, b0, v1, g1, b1, v2, g2, b2, fc_w, fc_b,
             *, batch_block=_BB, mxu_dtype=jnp.bfloat16):
    N, Cin, H, W = x_nchw.shape
    B = batch_block

    x = jnp.transpose(x_nchw, (0, 2, 3, 1)).reshape(N, H, W * Cin)
    n_pad = (-N) % B
    if n_pad:
        x = jnp.concatenate(
            [x, jnp.zeros((n_pad, H, W * Cin), x.dtype)], axis=0)
    Np = N + n_pad

    c1, c2, c3 = v0.shape[0], v1.shape[0], v2.shape[0]
    w1, w2, w3 = W, W // 2, W // 4
    r1 = _band_rhs(v0, g0, w1, mxu_dtype)
    r2 = _band_rhs(v1, g1, w2, mxu_dtype)
    r3 = _band_rhs(v2, g2, w3, mxu_dtype)
    b1t = jnp.tile(b0.astype(jnp.float32), w1).reshape(1, w1 * c1)
    b2t = jnp.tile(b1.astype(jnp.float32), w2).reshape(1, w2 * c2)
    b3t = jnp.tile(b2.astype(jnp.float32), w3).reshape(1, w3 * c3)

    nc = fc_w.shape[0]
    nc_pad = max(128, ((nc + 127) // 128) * 128)
    w4 = W // 8
    fp = jnp.zeros((c3, nc_pad), mxu_dtype).at[:, :nc].set(
        fc_w.T.astype(mxu_dtype))
    fw = jnp.tile(fp, (w4, 1))
    fb = jnp.zeros((1, nc_pad), jnp.float32).at[:, :nc].set(
        fc_b.astype(jnp.float32))

    h2, h3 = H // 2, H // 4
    out = pl.pallas_call(
        _fused_body,
        out_shape=jax.ShapeDtypeStruct((Np, nc_pad), jnp.float32),
        grid_spec=pltpu.PrefetchScalarGridSpec(
            num_scalar_prefetch=0,
            grid=(Np // B,),
            in_specs=[
                pl.BlockSpec((B, H, W * Cin), lambda n: (n, 0, 0)),
                pl.BlockSpec(r1.shape, lambda n: (0, 0)),
                pl.BlockSpec(b1t.shape, lambda n: (0, 0)),
                pl.BlockSpec(r2.shape, lambda n: (0, 0)),
                pl.BlockSpec(b2t.shape, lambda n: (0, 0)),
                pl.BlockSpec(r3.shape, lambda n: (0, 0)),
                pl.BlockSpec(b3t.shape, lambda n: (0, 0)),
                pl.BlockSpec(fw.shape, lambda n: (0, 0)),
                pl.BlockSpec(fb.shape, lambda n: (0, 0)),
            ],
            out_specs=pl.BlockSpec((B, nc_pad), lambda n: (n, 0)),
            scratch_shapes=[
                pltpu.VMEM((B, H + 2, W * Cin), jnp.float32),
                pltpu.VMEM((B, H, 3 * W * Cin), mxu_dtype),
                pltpu.VMEM((B, H, (w2 * c1) // 128, 128),
                           jnp.float32),
                pltpu.VMEM((B, h2 + 2, w2 * c1), jnp.float32),
                pltpu.VMEM((B, h2, 3 * w2 * c1), mxu_dtype),
                pltpu.VMEM((B, h2, (w3 * c2) // 128, 128),
                           jnp.float32),
                pltpu.VMEM((B, h3 + 2, w3 * c2), jnp.float32),
                pltpu.VMEM((B, h3, 3 * w3 * c2), mxu_dtype),
                pltpu.VMEM((B, h3, ((W // 8) * c3) // 128, 128),
                           jnp.float32),
            ],
        ),
        compiler_params=pltpu.CompilerParams(
            dimension_semantics=("parallel",)),
    )(x, r1, b1t, r2, b2t, r3, b3t, fw, fb)
    return out[:N, :nc]


def kernel(x, v0, g0, b0, v1, g1, b1, v2, g2, b2, fc_w, fc_b):
    return _forward(x, v0, g0, b0, v1, g1, b1, v2, g2, b2, fc_w, fc_b)
